# baseline (speedup 1.0000x reference)
import functools

import jax
import jax.numpy as jnp
from jax import lax
from jax.experimental import pallas as pl
from jax.experimental.pallas import tpu as pltpu
from jax.experimental.pallas import tpu_sc as plsc

NC = 2
NS = 16
CH = 128


def _sc_mesh():
    return plsc.VectorSubcoreMesh(core_axis_name="c", subcore_axis_name="s")


_SC_PARAMS = pltpu.CompilerParams(use_tc_tiling_on_sc=False)
_SC_PARAMS_SMALL_SCRATCH = pltpu.CompilerParams(
    use_tc_tiling_on_sc=False, internal_scratch_in_bytes=256 * 1024)


def _count_kernel(n_pad, n_chunks):
    rt = n_pad // NS

    @functools.partial(
        pl.kernel,
        out_type=jax.ShapeDtypeStruct((NC, n_pad, 16), jnp.float32),
        mesh=_sc_mesh(),
        scratch_types=[
            pltpu.VMEM_SHARED((n_pad, 16), jnp.float32),
            pltpu.VMEM((n_chunks, CH), jnp.int32),
            pltpu.VMEM((CH, 16), jnp.float32),
            pltpu.SemaphoreType.DMA,
        ],
        compiler_params=_SC_PARAMS,
    )
    def k(dst_hbm, out_hbm, acc_sh, didx_v, ones_v, sem):
        core = lax.axis_index("c")
        sub = lax.axis_index("s")
        wid = core * NS + sub
        row0 = sub * rt

        @pl.loop(0, CH)
        def _(i):
            ones_v[i, :] = jnp.zeros((16,), jnp.float32)

        idx_load = pltpu.async_copy(
            dst_hbm.at[pl.ds(wid * n_chunks, n_chunks)], didx_v, sem)

        @pl.loop(0, rt // CH)
        def _(b):
            pltpu.sync_copy(ones_v, acc_sh.at[pl.ds(row0 + b * CH, CH)])

        @pl.loop(0, CH)
        def _(i):
            ones_v[i, :] = jnp.ones((16,), jnp.float32)

        idx_load.wait()
        plsc.subcore_barrier()

        @pl.loop(0, n_chunks, step=8)
        def _(c):
            hs = [pltpu.async_copy(ones_v, acc_sh.at[didx_v.at[c + j]], sem,
                                   add=True)
                  for j in range(8)]
            for h in hs:
                h.wait()

        plsc.subcore_barrier()
        pltpu.sync_copy(acc_sh.at[pl.ds(row0, rt)], out_hbm.at[core, pl.ds(row0, rt)])

    return k


NB = 4


def _agg_kernel(n_pad, d, c0, c1, stage=True, nb=NB, ch=CH):
    rt = n_pad // NS

    @functools.partial(
        pl.kernel,
        out_type=jax.ShapeDtypeStruct((NC, n_pad, d), jnp.float32),
        mesh=_sc_mesh(),
        scratch_types=[
            pltpu.VMEM_SHARED((n_pad, d), jnp.float32),
            pltpu.VMEM_SHARED((n_pad, d) if stage else (8, d), jnp.float32),
            pltpu.VMEM((nb, ch), jnp.int32),
            pltpu.VMEM((nb, ch), jnp.int32),
        ] + [pltpu.VMEM((ch, d), jnp.float32)] * nb
          + [pltpu.SemaphoreType.DMA] * (4 * nb + 1),
        compiler_params=_SC_PARAMS,
    )
    def k(g_hbm, src_hbm, dst_hbm, out_hbm, acc_sh, gtab_sh, sidx_v, didx_v,
          *bufs):
        rows = bufs[:nb]
        gsem = bufs[nb:2 * nb]
        ssem = bufs[2 * nb:3 * nb]
        isem = bufs[3 * nb:4 * nb]
        jsem = bufs[4 * nb:5 * nb]
        tsem = bufs[5 * nb]
        core = lax.axis_index("c")
        sub = lax.axis_index("s")
        row0 = sub * rt

        def run(base, nch, do_stage):
            if do_stage:
                st = pltpu.async_copy(
                    g_hbm.at[pl.ds(row0, rt)], gtab_sh.at[pl.ds(row0, rt)],
                    tsem)
                src_tab = gtab_sh
            else:
                src_tab = g_hbm

            @pl.loop(0, ch)
            def _(i):
                @pl.loop(0, d // 16)
                def _(j):
                    rows[0][i, pl.ds(j * 16, 16)] = jnp.zeros((16,), jnp.float32)

            @pl.loop(0, rt // ch)
            def _(b):
                pltpu.sync_copy(rows[0], acc_sh.at[pl.ds(row0 + b * ch, ch)])

            if do_stage:
                st.wait()
            plsc.subcore_barrier()

            iloads = [pltpu.async_copy(src_hbm.at[base + b], sidx_v.at[b],
                                       isem[b]) for b in range(nb)]
            jloads = [pltpu.async_copy(dst_hbm.at[base + b], didx_v.at[b],
                                       jsem[b]) for b in range(nb)]
            for b in range(nb):
                iloads[b].wait()
                pltpu.async_copy(src_tab.at[sidx_v.at[b]], rows[b], gsem[b])

            @pl.loop(0, nch, step=nb)
            def _(c):
                adds = []
                for b in range(nb):
                    pltpu.make_async_copy(src_tab.at[sidx_v.at[b]], rows[b],
                                          gsem[b]).wait()
                    pltpu.make_async_copy(dst_hbm.at[base + c + b],
                                          didx_v.at[b], jsem[b]).wait()
                    adds.append(pltpu.async_copy(
                        rows[b], acc_sh.at[didx_v.at[b]], ssem[b], add=True))

                    @pl.when(c + b + nb < nch)
                    def _():
                        pltpu.async_copy(src_hbm.at[base + c + b + nb],
                                         sidx_v.at[b], isem[b])
                for b in range(nb):
                    adds[b].wait()

                    @pl.when(c + b + nb < nch)
                    def _():
                        pltpu.async_copy(dst_hbm.at[base + c + b + nb],
                                         didx_v.at[b], jsem[b])
                        pltpu.make_async_copy(src_hbm.at[base + c + b + nb],
                                              sidx_v.at[b], isem[b]).wait()
                        pltpu.async_copy(src_tab.at[sidx_v.at[b]], rows[b],
                                         gsem[b])

            plsc.subcore_barrier()
            pltpu.sync_copy(acc_sh.at[pl.ds(row0, rt)],
                            out_hbm.at[core, pl.ds(row0, rt)])

        @pl.when(core == 0)
        def _():
            run(sub * c0, c0, False)

        @pl.when(core == 1)
        def _():
            run(NS * c0 + sub * c1, c1, stage)

    return k


def _matmul_split(feat, loc, wa, wb, n_pad, bn):
    d = wa.shape[1]

    def body(f_ref, l_ref, wa_ref, wb_ref, o_ref):
        o_ref[...] = (
            jnp.dot(f_ref[...], wa_ref[...], preferred_element_type=jnp.float32)
            + jnp.dot(l_ref[...], wb_ref[...],
                      preferred_element_type=jnp.float32))

    return pl.pallas_call(
        body,
        grid=(n_pad // bn,),
        in_specs=[pl.BlockSpec((bn, feat.shape[1]), lambda i: (i, 0)),
                  pl.BlockSpec((bn, loc.shape[1]), lambda i: (i, 0)),
                  pl.BlockSpec(wa.shape, lambda i: (0, 0)),
                  pl.BlockSpec(wb.shape, lambda i: (0, 0))],
        out_specs=pl.BlockSpec((bn, d), lambda i: (i, 0)),
        out_shape=jax.ShapeDtypeStruct((n_pad, d), jnp.float32),
    )(feat, loc, wa, wb)


def _scale(h, cnt, bn):
    n_pad, d = h.shape

    def body(h_ref, cnt_ref, o_ref, d_ref):
        deg = 1.0 + cnt_ref[0, :, 0:1] + cnt_ref[1, :, 0:1]
        dis = lax.rsqrt(deg)
        o_ref[...] = h_ref[...] * dis
        d_ref[...] = jnp.broadcast_to(dis, d_ref.shape)

    return pl.pallas_call(
        body,
        grid=(n_pad // bn,),
        in_specs=[pl.BlockSpec((bn, d), lambda i: (i, 0)),
                  pl.BlockSpec((NC, bn, 16), lambda i: (0, i, 0))],
        out_specs=[pl.BlockSpec((bn, d), lambda i: (i, 0)),
                   pl.BlockSpec((bn, 16), lambda i: (i, 0))],
        out_shape=[jax.ShapeDtypeStruct((n_pad, d), jnp.float32),
                   jax.ShapeDtypeStruct((n_pad, 16), jnp.float32)],
    )(h, cnt)


def _layer1_post(acc1, g1, dis16, loc, b1, bn):
    n_pad, d = g1.shape
    ldim = loc.shape[1]

    def body(a_ref, g_ref, dis_ref, loc_ref, b_ref, o_ref):
        dis = dis_ref[:, 0:1]
        pre = (a_ref[0] + a_ref[1] + g_ref[...]) * dis + b_ref[...]
        z = jnp.maximum(pre, 0.0)
        o_ref[...] = jnp.concatenate([z * dis, loc_ref[...] * dis], axis=1)

    return pl.pallas_call(
        body,
        grid=(n_pad // bn,),
        in_specs=[pl.BlockSpec((NC, bn, d), lambda i: (0, i, 0)),
                  pl.BlockSpec((bn, d), lambda i: (i, 0)),
                  pl.BlockSpec((bn, 16), lambda i: (i, 0)),
                  pl.BlockSpec((bn, ldim), lambda i: (i, 0)),
                  pl.BlockSpec((1, d), lambda i: (0, 0))],
        out_specs=pl.BlockSpec((bn, d + ldim), lambda i: (i, 0)),
        out_shape=jax.ShapeDtypeStruct((n_pad, d + ldim), jnp.float32),
    )(acc1, g1, dis16, loc, b1)


def _layer2_post(acc2, g2, dis16, w2, b2, n, bn):
    n_pad, d = g2.shape
    dout = w2.shape[1]

    def body(a_ref, g_ref, dis_ref, w_ref, b_ref, o_ref):
        dis = dis_ref[:, 0:1]
        u = (a_ref[0] + a_ref[1] + g_ref[...]) * dis
        o_ref[...] = jnp.dot(u, w_ref[...],
                             preferred_element_type=jnp.float32) + b_ref[...]

    return pl.pallas_call(
        body,
        grid=(n_pad // bn,),
        in_specs=[pl.BlockSpec((NC, bn, d), lambda i: (0, i, 0)),
                  pl.BlockSpec((bn, d), lambda i: (i, 0)),
                  pl.BlockSpec((bn, 16), lambda i: (i, 0)),
                  pl.BlockSpec((d, dout), lambda i: (0, 0)),
                  pl.BlockSpec((1, dout), lambda i: (0, 0))],
        out_specs=pl.BlockSpec((bn, dout), lambda i: (i, 0)),
        out_shape=jax.ShapeDtypeStruct((n, dout), jnp.float32),
    )(acc2, g2, dis16, w2, b2)


def kernel(edge_indices, features, location_embedding, W1, b1, W2, b2):
    n = features.shape[0]
    e = edge_indices.shape[1]

    npad_unit = NS * CH
    n_pad = -(-n // npad_unit) * npad_unit
    ep_unit = NC * NS * CH * 8
    e_pad = -(-e // ep_unit) * ep_unit
    n_chunks = e_pad // (NC * NS * CH)

    src = edge_indices[0]
    dst = edge_indices[1]
    pad = e_pad - e
    src_p = jnp.concatenate([src, jnp.zeros((pad,), jnp.int32)]).reshape(-1, CH)
    dst_p = jnp.concatenate([dst, jnp.full((pad,), n, jnp.int32)]).reshape(-1, CH)

    b1r = b1.reshape(1, -1)
    b2r = b2.reshape(1, -1)
    dfeat = features.shape[1]
    w1a = W1[:dfeat]
    w1b = W1[dfeat:]

    bn = n_pad // 8

    c_pair = e_pad // (CH * NS)
    c0b = (c_pair * 3 // 5 // NB) * NB
    c0b = min(max(c0b, NB), c_pair - NB)
    c1b = c_pair - c0b

    cnt = _count_kernel(n_pad, n_chunks)(dst_p)
    h1 = _matmul_split(features, location_embedding, w1a, w1b, n_pad, bn)
    g1, dis16 = _scale(h1, cnt, bn)
    acc1 = _agg_kernel(n_pad, h1.shape[1], c0b, c1b, stage=True)(
        g1, src_p, dst_p)
    g2 = _layer1_post(acc1, g1, dis16, location_embedding, b1r, bn)
    c_pair2 = e_pad // (64 * NS)
    c0b2 = (c_pair2 * 31 // 64 // NB) * NB
    c1b2 = c_pair2 - c0b2
    acc2 = _agg_kernel(n_pad, g2.shape[1], c0b2, c1b2, stage=True, nb=NB,
                       ch=64)(g2, src_p.reshape(-1, 64), dst_p.reshape(-1, 64))
    return _layer2_post(acc2, g2, dis16, W2, b2r, n, bn)


# speedup vs baseline: 1.0453x; 1.0453x over previous
import functools

import jax
import jax.numpy as jnp
from jax import lax
from jax.experimental import pallas as pl
from jax.experimental.pallas import tpu as pltpu
from jax.experimental.pallas import tpu_sc as plsc

NC = 2
NS = 16
CH = 128


def _sc_mesh():
    return plsc.VectorSubcoreMesh(core_axis_name="c", subcore_axis_name="s")


_SC_PARAMS = pltpu.CompilerParams(use_tc_tiling_on_sc=False)
_SC_PARAMS_SMALL_SCRATCH = pltpu.CompilerParams(
    use_tc_tiling_on_sc=False, internal_scratch_in_bytes=256 * 1024)


def _count_kernel(n_pad, cc0, cc1):
    rt = n_pad // NS
    cmax = max(cc0, cc1)

    @functools.partial(
        pl.kernel,
        out_type=jax.ShapeDtypeStruct((NC, n_pad, 16), jnp.float32),
        mesh=_sc_mesh(),
        scratch_types=[
            pltpu.VMEM_SHARED((n_pad, 16), jnp.float32),
            pltpu.VMEM((cmax, CH), jnp.int32),
            pltpu.VMEM((CH, 16), jnp.float32),
            pltpu.SemaphoreType.DMA,
        ],
        compiler_params=_SC_PARAMS,
    )
    def k(dst_hbm, out_hbm, acc_sh, didx_v, ones_v, sem):
        core = lax.axis_index("c")
        sub = lax.axis_index("s")
        row0 = sub * rt

        def runc(base, nch):
            @pl.loop(0, CH)
            def _(i):
                ones_v[i, :] = jnp.zeros((16,), jnp.float32)

            idx_load = pltpu.async_copy(
                dst_hbm.at[pl.ds(base, nch)], didx_v.at[pl.ds(0, nch)], sem)

            @pl.loop(0, rt // CH)
            def _(b):
                pltpu.sync_copy(ones_v, acc_sh.at[pl.ds(row0 + b * CH, CH)])

            @pl.loop(0, CH)
            def _(i):
                ones_v[i, :] = jnp.ones((16,), jnp.float32)

            idx_load.wait()
            plsc.subcore_barrier()

            @pl.loop(0, nch, step=8)
            def _(c):
                hs = [pltpu.async_copy(ones_v, acc_sh.at[didx_v.at[c + j]],
                                       sem, add=True)
                      for j in range(8)]
                for h in hs:
                    h.wait()

            plsc.subcore_barrier()
            pltpu.sync_copy(acc_sh.at[pl.ds(row0, rt)],
                            out_hbm.at[core, pl.ds(row0, rt)])

        @pl.when(core == 0)
        def _():
            runc(sub * cc0, cc0)

        @pl.when(core == 1)
        def _():
            runc(NS * cc0 + sub * cc1, cc1)

    return k


NB = 4


def _agg_kernel(n_pad, d, c0, c1, stage=True, nb=NB, ch=CH):
    rt = n_pad // NS

    @functools.partial(
        pl.kernel,
        out_type=jax.ShapeDtypeStruct((NC, n_pad, d), jnp.float32),
        mesh=_sc_mesh(),
        scratch_types=[
            pltpu.VMEM_SHARED((n_pad, d), jnp.float32),
            pltpu.VMEM_SHARED((n_pad, d) if stage else (8, d), jnp.float32),
            pltpu.VMEM((nb, ch), jnp.int32),
            pltpu.VMEM((nb, ch), jnp.int32),
        ] + [pltpu.VMEM((ch, d), jnp.float32)] * nb
          + [pltpu.SemaphoreType.DMA] * (4 * nb + 1),
        compiler_params=_SC_PARAMS,
    )
    def k(g_hbm, src_hbm, dst_hbm, out_hbm, acc_sh, gtab_sh, sidx_v, didx_v,
          *bufs):
        rows = bufs[:nb]
        gsem = bufs[nb:2 * nb]
        ssem = bufs[2 * nb:3 * nb]
        isem = bufs[3 * nb:4 * nb]
        jsem = bufs[4 * nb:5 * nb]
        tsem = bufs[5 * nb]
        core = lax.axis_index("c")
        sub = lax.axis_index("s")
        row0 = sub * rt

        def run(base, nch, do_stage):
            if do_stage:
                st = pltpu.async_copy(
                    g_hbm.at[pl.ds(row0, rt)], gtab_sh.at[pl.ds(row0, rt)],
                    tsem)
                src_tab = gtab_sh
            else:
                src_tab = g_hbm

            @pl.loop(0, ch)
            def _(i):
                @pl.loop(0, d // 16)
                def _(j):
                    rows[0][i, pl.ds(j * 16, 16)] = jnp.zeros((16,), jnp.float32)

            @pl.loop(0, rt // ch)
            def _(b):
                pltpu.sync_copy(rows[0], acc_sh.at[pl.ds(row0 + b * ch, ch)])

            if do_stage:
                st.wait()
            plsc.subcore_barrier()

            iloads = [pltpu.async_copy(src_hbm.at[base + b], sidx_v.at[b],
                                       isem[b]) for b in range(nb)]
            jloads = [pltpu.async_copy(dst_hbm.at[base + b], didx_v.at[b],
                                       jsem[b]) for b in range(nb)]
            for b in range(nb):
                iloads[b].wait()
                pltpu.async_copy(src_tab.at[sidx_v.at[b]], rows[b], gsem[b])

            @pl.loop(0, nch, step=nb)
            def _(c):
                adds = []
                for b in range(nb):
                    pltpu.make_async_copy(src_tab.at[sidx_v.at[b]], rows[b],
                                          gsem[b]).wait()
                    pltpu.make_async_copy(dst_hbm.at[base + c + b],
                                          didx_v.at[b], jsem[b]).wait()
                    adds.append(pltpu.async_copy(
                        rows[b], acc_sh.at[didx_v.at[b]], ssem[b], add=True))

                    @pl.when(c + b + nb < nch)
                    def _():
                        pltpu.async_copy(src_hbm.at[base + c + b + nb],
                                         sidx_v.at[b], isem[b])
                for b in range(nb):
                    adds[b].wait()

                    @pl.when(c + b + nb < nch)
                    def _():
                        pltpu.async_copy(dst_hbm.at[base + c + b + nb],
                                         didx_v.at[b], jsem[b])
                        pltpu.make_async_copy(src_hbm.at[base + c + b + nb],
                                              sidx_v.at[b], isem[b]).wait()
                        pltpu.async_copy(src_tab.at[sidx_v.at[b]], rows[b],
                                         gsem[b])

            plsc.subcore_barrier()
            pltpu.sync_copy(acc_sh.at[pl.ds(row0, rt)],
                            out_hbm.at[core, pl.ds(row0, rt)])

        @pl.when(core == 0)
        def _():
            run(sub * c0, c0, False)

        @pl.when(core == 1)
        def _():
            run(NS * c0 + sub * c1, c1, stage)

    return k


def _matmul_split(feat, loc, wa, wb, n_pad, bn):
    d = wa.shape[1]

    def body(f_ref, l_ref, wa_ref, wb_ref, o_ref):
        o_ref[...] = (
            jnp.dot(f_ref[...], wa_ref[...], preferred_element_type=jnp.float32)
            + jnp.dot(l_ref[...], wb_ref[...],
                      preferred_element_type=jnp.float32))

    return pl.pallas_call(
        body,
        grid=(n_pad // bn,),
        in_specs=[pl.BlockSpec((bn, feat.shape[1]), lambda i: (i, 0)),
                  pl.BlockSpec((bn, loc.shape[1]), lambda i: (i, 0)),
                  pl.BlockSpec(wa.shape, lambda i: (0, 0)),
                  pl.BlockSpec(wb.shape, lambda i: (0, 0))],
        out_specs=pl.BlockSpec((bn, d), lambda i: (i, 0)),
        out_shape=jax.ShapeDtypeStruct((n_pad, d), jnp.float32),
    )(feat, loc, wa, wb)


def _scale(h, cnt, bn):
    n_pad, d = h.shape

    def body(h_ref, cnt_ref, o_ref, d_ref):
        deg = 1.0 + cnt_ref[0, :, 0:1] + cnt_ref[1, :, 0:1]
        dis = lax.rsqrt(deg)
        o_ref[...] = h_ref[...] * dis
        d_ref[...] = jnp.broadcast_to(dis, d_ref.shape)

    return pl.pallas_call(
        body,
        grid=(n_pad // bn,),
        in_specs=[pl.BlockSpec((bn, d), lambda i: (i, 0)),
                  pl.BlockSpec((NC, bn, 16), lambda i: (0, i, 0))],
        out_specs=[pl.BlockSpec((bn, d), lambda i: (i, 0)),
                   pl.BlockSpec((bn, 16), lambda i: (i, 0))],
        out_shape=[jax.ShapeDtypeStruct((n_pad, d), jnp.float32),
                   jax.ShapeDtypeStruct((n_pad, 16), jnp.float32)],
    )(h, cnt)


def _layer1_post(acc1, g1, dis16, loc, b1, bn):
    n_pad, d = g1.shape
    ldim = loc.shape[1]

    def body(a_ref, g_ref, dis_ref, loc_ref, b_ref, o_ref):
        dis = dis_ref[:, 0:1]
        pre = (a_ref[0] + a_ref[1] + g_ref[...]) * dis + b_ref[...]
        z = jnp.maximum(pre, 0.0)
        o_ref[...] = jnp.concatenate([z * dis, loc_ref[...] * dis], axis=1)

    return pl.pallas_call(
        body,
        grid=(n_pad // bn,),
        in_specs=[pl.BlockSpec((NC, bn, d), lambda i: (0, i, 0)),
                  pl.BlockSpec((bn, d), lambda i: (i, 0)),
                  pl.BlockSpec((bn, 16), lambda i: (i, 0)),
                  pl.BlockSpec((bn, ldim), lambda i: (i, 0)),
                  pl.BlockSpec((1, d), lambda i: (0, 0))],
        out_specs=pl.BlockSpec((bn, d + ldim), lambda i: (i, 0)),
        out_shape=jax.ShapeDtypeStruct((n_pad, d + ldim), jnp.float32),
    )(acc1, g1, dis16, loc, b1)


def _layer2_post(acc2, g2, dis16, w2, b2, n, bn):
    n_pad, d = g2.shape
    dout = w2.shape[1]

    def body(a_ref, g_ref, dis_ref, w_ref, b_ref, o_ref):
        dis = dis_ref[:, 0:1]
        u = (a_ref[0] + a_ref[1] + g_ref[...]) * dis
        o_ref[...] = jnp.dot(u, w_ref[...],
                             preferred_element_type=jnp.float32) + b_ref[...]

    return pl.pallas_call(
        body,
        grid=(n_pad // bn,),
        in_specs=[pl.BlockSpec((NC, bn, d), lambda i: (0, i, 0)),
                  pl.BlockSpec((bn, d), lambda i: (i, 0)),
                  pl.BlockSpec((bn, 16), lambda i: (i, 0)),
                  pl.BlockSpec((d, dout), lambda i: (0, 0)),
                  pl.BlockSpec((1, dout), lambda i: (0, 0))],
        out_specs=pl.BlockSpec((bn, dout), lambda i: (i, 0)),
        out_shape=jax.ShapeDtypeStruct((n, dout), jnp.float32),
    )(acc2, g2, dis16, w2, b2)


def kernel(edge_indices, features, location_embedding, W1, b1, W2, b2):
    n = features.shape[0]
    e = edge_indices.shape[1]

    npad_unit = NS * CH
    n_pad = -(-n // npad_unit) * npad_unit
    ep_unit = NC * NS * CH * 8
    e_pad = -(-e // ep_unit) * ep_unit
    n_chunks = e_pad // (NC * NS * CH)

    src = edge_indices[0]
    dst = edge_indices[1]
    pad = e_pad - e
    src_p = jnp.concatenate([src, jnp.zeros((pad,), jnp.int32)]).reshape(-1, CH)
    dst_p = jnp.concatenate([dst, jnp.full((pad,), n, jnp.int32)]).reshape(-1, CH)

    b1r = b1.reshape(1, -1)
    b2r = b2.reshape(1, -1)
    dfeat = features.shape[1]
    w1a = W1[:dfeat]
    w1b = W1[dfeat:]

    bn = n_pad // 8

    c_pair = e_pad // (CH * NS)
    c0b = (c_pair * 3 // 5 // NB) * NB
    c0b = min(max(c0b, NB), c_pair - NB)
    c1b = c_pair - c0b

    cc0 = (c_pair * 3 // 5 // 8) * 8
    cc0 = min(max(cc0, 8), c_pair - 8)
    cc1 = c_pair - cc0
    cnt = _count_kernel(n_pad, cc0, cc1)(dst_p)
    h1 = _matmul_split(features, location_embedding, w1a, w1b, n_pad, bn)
    g1, dis16 = _scale(h1, cnt, bn)
    acc1 = _agg_kernel(n_pad, h1.shape[1], c0b, c1b, stage=True)(
        g1, src_p, dst_p)
    g2 = _layer1_post(acc1, g1, dis16, location_embedding, b1r, bn)
    c0b2 = (c_pair * 31 // 64 // 2) * 2
    c1b2 = c_pair - c0b2
    acc2 = _agg_kernel(n_pad, g2.shape[1], c0b2, c1b2, stage=True, nb=2)(
        g2, src_p, dst_p)
    return _layer2_post(acc2, g2, dis16, W2, b2r, n, bn)
